# baseline (speedup 1.0000x reference)
import functools

import jax
import jax.numpy as jnp
from jax import lax
from jax.experimental import pallas as pl
from jax.experimental.pallas import tpu as pltpu
from jax.experimental.pallas import tpu_sc as plsc

_N = 10000
_E = 320000
_D = 128
_B = 1024
_NEG = 10

_NS = 16
_CHUNK = 128
_N_PAD = 10240
_E_PAD = 327680
_NCHUNK = _E_PAD // _CHUNK
_N0 = 80
_N1 = 80
_STRIPE = _N_PAD // _NS
_SUP = 8
_G = 4096
_GCHUNK = 128
_G0 = 28

_mesh = plsc.VectorSubcoreMesh(core_axis_name="c", subcore_axis_name="s")


@functools.partial(
    pl.kernel,
    out_type=(
        jax.ShapeDtypeStruct((_N_PAD, _D), jnp.float32),
        jax.ShapeDtypeStruct((_N_PAD, _D), jnp.float32),
    ),
    mesh=_mesh,
    scratch_types=[
        pltpu.VMEM((_SUP, _CHUNK), jnp.int32),
        pltpu.VMEM((_SUP, _CHUNK), jnp.int32),
        pltpu.VMEM((2, _CHUNK, _D), jnp.float32),
        pltpu.VMEM_SHARED((_N_PAD, _D), jnp.float32),
        pltpu.SemaphoreType.DMA,
        pltpu.SemaphoreType.DMA,
    ],
)
def _segsum(table_hbm, src_hbm, dst_hbm, out0_hbm, out1_hbm,
            src_v, dst_v, rows_v, acc, sem_a, sem_b):
    c = lax.axis_index("c")
    s = lax.axis_index("s")
    def zrow(i, carry):
        for k in range(_D // 16):
            rows_v[0, i, pl.ds(16 * k, 16)] = jnp.zeros((16,), jnp.float32)
        return carry

    lax.fori_loop(0, _CHUNK, zrow, 0)
    for m in range(_STRIPE // _CHUNK):
        pltpu.sync_copy(rows_v.at[0],
                        acc.at[pl.ds(s * _STRIPE + m * _CHUNK, _CHUNK)])
    plsc.subcore_barrier()

    base_chunk = jnp.where(c == 0, s * _N0, _NS * _N0 + s * _N1)
    nblocks = jnp.where(c == 0, _N0 // _SUP, _N1 // _SUP)

    def outer(t, carry):
        blk = base_chunk + t * _SUP
        pltpu.sync_copy(src_hbm.at[pl.ds(blk, _SUP)], src_v)
        pltpu.sync_copy(dst_hbm.at[pl.ds(blk, _SUP)], dst_v)

        def body(i, carry2):
            j = 2 * i
            cp_a = pltpu.async_copy(table_hbm.at[src_v.at[j]], rows_v.at[0],
                                    sem_a)
            cp_b = pltpu.async_copy(table_hbm.at[src_v.at[j + 1]],
                                    rows_v.at[1], sem_b)
            cp_a.wait()
            pltpu.sync_copy(rows_v.at[0], acc.at[dst_v.at[j]], add=True)
            cp_b.wait()
            pltpu.sync_copy(rows_v.at[1], acc.at[dst_v.at[j + 1]], add=True)
            return carry2

        lax.fori_loop(0, _SUP // 2, body, 0)
        return carry

    lax.fori_loop(0, nblocks, outer, 0)
    plsc.subcore_barrier()

    @pl.when(c == 0)
    def _():
        pltpu.sync_copy(acc.at[pl.ds(s * _STRIPE, _STRIPE)],
                        out0_hbm.at[pl.ds(s * _STRIPE, _STRIPE)])

    @pl.when(c == 1)
    def _():
        pltpu.sync_copy(acc.at[pl.ds(s * _STRIPE, _STRIPE)],
                        out1_hbm.at[pl.ds(s * _STRIPE, _STRIPE)])


@functools.partial(
    pl.kernel,
    out_type=(
        jax.ShapeDtypeStruct((_G, _D), jnp.float32),
        jax.ShapeDtypeStruct((2, _G, _D), jnp.float32),
        jax.ShapeDtypeStruct((2, _N_PAD, _D), jnp.float32),
    ),
    mesh=_mesh,
    scratch_types=[
        pltpu.VMEM((_SUP, _CHUNK), jnp.int32),
        pltpu.VMEM((_SUP, _CHUNK), jnp.int32),
        pltpu.VMEM((2, _CHUNK, _D), jnp.float32),
        pltpu.VMEM((2, _GCHUNK), jnp.int32),
        pltpu.VMEM_SHARED((_N_PAD, _D), jnp.float32),
        pltpu.SemaphoreType.DMA,
        pltpu.SemaphoreType.DMA,
    ],
)
def _segsum_gather(table_hbm, src_hbm, dst_hbm, feat_hbm, idx_hbm,
                   gf_hbm, gp_hbm, q_hbm,
                   src_v, dst_v, rows_v, gidx_v, acc, sem_a, sem_b):
    c = lax.axis_index("c")
    s = lax.axis_index("s")

    def zrow(i, carry):
        for k in range(_D // 16):
            rows_v[0, i, pl.ds(16 * k, 16)] = jnp.zeros((16,), jnp.float32)
        return carry

    lax.fori_loop(0, _CHUNK, zrow, 0)
    for m in range(_STRIPE // _CHUNK):
        pltpu.sync_copy(rows_v.at[0],
                        acc.at[pl.ds(s * _STRIPE + m * _CHUNK, _CHUNK)])
    pltpu.sync_copy(idx_hbm.at[s], gidx_v)
    plsc.subcore_barrier()

    base_chunk = jnp.where(c == 0, s * _N0, _NS * _N0 + s * _N1)
    nblocks = jnp.where(c == 0, _N0 // _SUP, _N1 // _SUP)

    def outer(t, carry):
        blk = base_chunk + t * _SUP
        pltpu.sync_copy(src_hbm.at[pl.ds(blk, _SUP)], src_v)
        pltpu.sync_copy(dst_hbm.at[pl.ds(blk, _SUP)], dst_v)

        def body(i, carry2):
            j = 2 * i
            cp_a = pltpu.async_copy(table_hbm.at[src_v.at[j]], rows_v.at[0],
                                    sem_a)
            cp_b = pltpu.async_copy(table_hbm.at[src_v.at[j + 1]],
                                    rows_v.at[1], sem_b)
            cp_a.wait()
            pltpu.sync_copy(rows_v.at[0], acc.at[dst_v.at[j]], add=True)
            cp_b.wait()
            pltpu.sync_copy(rows_v.at[1], acc.at[dst_v.at[j + 1]], add=True)
            return carry2

        lax.fori_loop(0, _SUP // 2, body, 0)
        return carry

    lax.fori_loop(0, nblocks, outer, 0)
    plsc.subcore_barrier()

    pltpu.sync_copy(acc.at[pl.ds(s * _STRIPE, _STRIPE)],
                    q_hbm.at[c, pl.ds(s * _STRIPE, _STRIPE)])
    fg = pltpu.async_copy(feat_hbm.at[gidx_v.at[c]], rows_v.at[0], sem_a)
    fg.wait()
    pltpu.sync_copy(rows_v.at[0],
                    gf_hbm.at[pl.ds((2 * s + c) * _GCHUNK, _GCHUNK)])
    plsc.subcore_barrier()
    ga0 = pltpu.async_copy(q_hbm.at[c].at[gidx_v.at[0]], rows_v.at[0], sem_a)
    ga1 = pltpu.async_copy(q_hbm.at[c].at[gidx_v.at[1]], rows_v.at[1], sem_b)
    ga0.wait()
    wr0 = pltpu.async_copy(rows_v.at[0],
                           gp_hbm.at[c, pl.ds(2 * s * _GCHUNK, _GCHUNK)],
                           sem_a)
    ga1.wait()
    wr1 = pltpu.async_copy(rows_v.at[1],
                           gp_hbm.at[c, pl.ds((2 * s + 1) * _GCHUNK, _GCHUNK)],
                           sem_b)
    wr0.wait()
    wr1.wait()


def _layer1_body(feat_ref, p0_ref, p1_ref, w1a_ref, w1b_ref, b1_ref, out_ref):
    ns = p0_ref[...] + p1_ref[...]
    acc = jnp.dot(feat_ref[...], w1a_ref[...],
                  preferred_element_type=jnp.float32)
    acc = acc + jnp.dot(ns, w1b_ref[...], preferred_element_type=jnp.float32)
    out_ref[...] = jnp.tanh(acc + b1_ref[...])


def _layer2_body(feat_ref, p0_ref, p1_ref, w2a_ref, w2b_ref, b2_ref, out_ref):
    ns = p0_ref[...] + p1_ref[...]
    h2 = jnp.dot(feat_ref[...], w2a_ref[...],
                 preferred_element_type=jnp.float32)
    h2 = h2 + jnp.dot(ns, w2b_ref[...], preferred_element_type=jnp.float32)
    h2 = h2 + b2_ref[...]
    nrm = jnp.sqrt(jnp.sum(h2 * h2, axis=1, keepdims=True))
    out_ref[...] = h2 / jnp.maximum(nrm, 1e-12)


_LBLK = 1024


def _layer(body, feat_pad, p0, p1, wa, wb, br):
    grid = (_N_PAD // _LBLK,)
    row_spec = pl.BlockSpec((_LBLK, _D), lambda i: (i, 0))
    full_spec = pl.BlockSpec((_D, _D), lambda i: (0, 0))
    bias_spec = pl.BlockSpec((1, _D), lambda i: (0, 0))
    return pl.pallas_call(
        body,
        grid=grid,
        in_specs=[row_spec, row_spec, row_spec, full_spec, full_spec,
                  bias_spec],
        out_specs=row_spec,
        out_shape=jax.ShapeDtypeStruct((_N_PAD, _D), jnp.float32),
    )(feat_pad, p0, p1, wa, wb, br)


def kernel(feat_data, edge_index, inputs1, inputs2, neg, W1, b1, W2, b2):
    feat_pad = jnp.zeros((_N_PAD, _D), jnp.float32).at[:_N].set(feat_data)
    src = edge_index[0].astype(jnp.int32)
    dst = edge_index[1].astype(jnp.int32)
    pad_e = _E_PAD - _E
    fill = jnp.arange(pad_e, dtype=jnp.int32)
    src_pad = jnp.concatenate(
        [src, fill % _N]).reshape(_NCHUNK, _CHUNK)
    dst_pad = jnp.concatenate(
        [dst, _N + fill % (_N_PAD - _N)]).reshape(_NCHUNK, _CHUNK)

    p0, p1 = _segsum(feat_pad, src_pad, dst_pad)

    w1t = W1.T
    h1 = _layer(_layer1_body, feat_pad, p0, p1, w1t[:_D], w1t[_D:],
                b1.reshape(1, _D))

    idx = jnp.concatenate([
        inputs1.astype(jnp.int32), inputs2.astype(jnp.int32),
        neg.astype(jnp.int32),
        jnp.zeros((_G - 2 * _B - _NEG,), jnp.int32)]).reshape(_NS, 2, _GCHUNK)
    gf, gp, _q = _segsum_gather(h1, src_pad, dst_pad, feat_pad, idx)
    g0, g1 = gp[0], gp[1]

    w2t = W2.T
    out = pl.pallas_call(
        _layer2_body,
        out_shape=jax.ShapeDtypeStruct((_G, _D), jnp.float32),
    )(gf, g0, g1, w2t[:_D], w2t[_D:], b2.reshape(1, _D))
    return (out[:_B], out[_B:2 * _B], out[2 * _B:2 * _B + _NEG])


# speedup vs baseline: 1.2046x; 1.2046x over previous
import functools

import jax
import jax.numpy as jnp
from jax import lax
from jax.experimental import pallas as pl
from jax.experimental.pallas import tpu as pltpu
from jax.experimental.pallas import tpu_sc as plsc

_N = 10000
_E = 320000
_D = 128
_B = 1024
_NEG = 10

_NS = 16
_CHUNK = 128
_N_PAD = 10240
_E_PAD = 327680
_NCHUNK = _E_PAD // _CHUNK
_N0 = 80
_N1 = 80
_STRIPE = _N_PAD // _NS
_SUP = 8
_G = 4096
_GCHUNK = 128
_G0 = 28

_mesh = plsc.VectorSubcoreMesh(core_axis_name="c", subcore_axis_name="s")


@functools.partial(
    pl.kernel,
    out_type=(
        jax.ShapeDtypeStruct((_N_PAD, _D), jnp.float32),
        jax.ShapeDtypeStruct((_N_PAD, _D), jnp.float32),
        jax.ShapeDtypeStruct((_G, _D), jnp.float32),
    ),
    mesh=_mesh,
    scratch_types=[
        pltpu.VMEM((_SUP, _CHUNK), jnp.int32),
        pltpu.VMEM((_SUP, _CHUNK), jnp.int32),
        pltpu.VMEM((2, _CHUNK, _D), jnp.float32),
        pltpu.VMEM((2, _GCHUNK), jnp.int32),
        pltpu.VMEM_SHARED((_N_PAD, _D), jnp.float32),
        pltpu.SemaphoreType.DMA,
        pltpu.SemaphoreType.DMA,
    ],
)
def _segsum(table_hbm, src_hbm, dst_hbm, idx_hbm, out0_hbm, out1_hbm, gf_hbm,
            src_v, dst_v, rows_v, gidx_v, acc, sem_a, sem_b):
    c = lax.axis_index("c")
    s = lax.axis_index("s")
    def zrow(i, carry):
        for k in range(_D // 16):
            rows_v[0, i, pl.ds(16 * k, 16)] = jnp.zeros((16,), jnp.float32)
        return carry

    lax.fori_loop(0, _CHUNK, zrow, 0)
    for m in range(_STRIPE // _CHUNK):
        pltpu.sync_copy(rows_v.at[0],
                        acc.at[pl.ds(s * _STRIPE + m * _CHUNK, _CHUNK)])
    pltpu.sync_copy(idx_hbm.at[s], gidx_v)
    plsc.subcore_barrier()

    base_chunk = jnp.where(c == 0, s * _N0, _NS * _N0 + s * _N1)
    nblocks = jnp.where(c == 0, _N0 // _SUP, _N1 // _SUP)

    def outer(t, carry):
        blk = base_chunk + t * _SUP
        pltpu.sync_copy(src_hbm.at[pl.ds(blk, _SUP)], src_v)
        pltpu.sync_copy(dst_hbm.at[pl.ds(blk, _SUP)], dst_v)

        def body(i, carry2):
            j = 2 * i
            cp_a = pltpu.async_copy(table_hbm.at[src_v.at[j]], rows_v.at[0],
                                    sem_a)
            cp_b = pltpu.async_copy(table_hbm.at[src_v.at[j + 1]],
                                    rows_v.at[1], sem_b)
            cp_a.wait()
            pltpu.sync_copy(rows_v.at[0], acc.at[dst_v.at[j]], add=True)
            cp_b.wait()
            pltpu.sync_copy(rows_v.at[1], acc.at[dst_v.at[j + 1]], add=True)
            return carry2

        lax.fori_loop(0, _SUP // 2, body, 0)
        return carry

    lax.fori_loop(0, nblocks, outer, 0)
    fg = pltpu.async_copy(table_hbm.at[gidx_v.at[c]], rows_v.at[0], sem_a)
    fg.wait()
    pltpu.sync_copy(rows_v.at[0],
                    gf_hbm.at[pl.ds((2 * s + c) * _GCHUNK, _GCHUNK)])
    plsc.subcore_barrier()

    @pl.when(c == 0)
    def _():
        pltpu.sync_copy(acc.at[pl.ds(s * _STRIPE, _STRIPE)],
                        out0_hbm.at[pl.ds(s * _STRIPE, _STRIPE)])

    @pl.when(c == 1)
    def _():
        pltpu.sync_copy(acc.at[pl.ds(s * _STRIPE, _STRIPE)],
                        out1_hbm.at[pl.ds(s * _STRIPE, _STRIPE)])


@functools.partial(
    pl.kernel,
    out_type=jax.ShapeDtypeStruct((2, _G, _D), jnp.float32),
    mesh=_mesh,
    scratch_types=[
        pltpu.VMEM((_SUP, _CHUNK), jnp.int32),
        pltpu.VMEM((_SUP, _CHUNK), jnp.int32),
        pltpu.VMEM((2, _CHUNK, _D), jnp.float32),
        pltpu.VMEM((2, _GCHUNK), jnp.int32),
        pltpu.VMEM_SHARED((_N_PAD, _D), jnp.float32),
        pltpu.SemaphoreType.DMA,
        pltpu.SemaphoreType.DMA,
    ],
)
def _segsum_gather(table_hbm, src_hbm, dst_hbm, idx_hbm, gp_hbm,
                   src_v, dst_v, rows_v, gidx_v, acc, sem_a, sem_b):
    c = lax.axis_index("c")
    s = lax.axis_index("s")

    def zrow(i, carry):
        for k in range(_D // 16):
            rows_v[0, i, pl.ds(16 * k, 16)] = jnp.zeros((16,), jnp.float32)
        return carry

    lax.fori_loop(0, _CHUNK, zrow, 0)
    for m in range(_STRIPE // _CHUNK):
        pltpu.sync_copy(rows_v.at[0],
                        acc.at[pl.ds(s * _STRIPE + m * _CHUNK, _CHUNK)])
    pltpu.sync_copy(idx_hbm.at[s], gidx_v)
    plsc.subcore_barrier()

    base_chunk = jnp.where(c == 0, s * _N0, _NS * _N0 + s * _N1)
    nblocks = jnp.where(c == 0, _N0 // _SUP, _N1 // _SUP)

    def outer(t, carry):
        blk = base_chunk + t * _SUP
        pltpu.sync_copy(src_hbm.at[pl.ds(blk, _SUP)], src_v)
        pltpu.sync_copy(dst_hbm.at[pl.ds(blk, _SUP)], dst_v)

        def body(i, carry2):
            j = 2 * i
            cp_a = pltpu.async_copy(table_hbm.at[src_v.at[j]], rows_v.at[0],
                                    sem_a)
            cp_b = pltpu.async_copy(table_hbm.at[src_v.at[j + 1]],
                                    rows_v.at[1], sem_b)
            cp_a.wait()
            pltpu.sync_copy(rows_v.at[0], acc.at[dst_v.at[j]], add=True)
            cp_b.wait()
            pltpu.sync_copy(rows_v.at[1], acc.at[dst_v.at[j + 1]], add=True)
            return carry2

        lax.fori_loop(0, _SUP // 2, body, 0)
        return carry

    lax.fori_loop(0, nblocks, outer, 0)
    plsc.subcore_barrier()

    ga0 = pltpu.async_copy(acc.at[gidx_v.at[0]], rows_v.at[0], sem_a)
    ga1 = pltpu.async_copy(acc.at[gidx_v.at[1]], rows_v.at[1], sem_b)
    ga0.wait()
    wr0 = pltpu.async_copy(rows_v.at[0],
                           gp_hbm.at[c, pl.ds(2 * s * _GCHUNK, _GCHUNK)],
                           sem_a)
    ga1.wait()
    wr1 = pltpu.async_copy(rows_v.at[1],
                           gp_hbm.at[c, pl.ds((2 * s + 1) * _GCHUNK, _GCHUNK)],
                           sem_b)
    wr0.wait()
    wr1.wait()


def _layer1_body(feat_ref, p0_ref, p1_ref, w1a_ref, w1b_ref, b1_ref, out_ref):
    ns = p0_ref[...] + p1_ref[...]
    acc = jnp.dot(feat_ref[...], w1a_ref[...],
                  preferred_element_type=jnp.float32)
    acc = acc + jnp.dot(ns, w1b_ref[...], preferred_element_type=jnp.float32)
    out_ref[...] = jnp.tanh(acc + b1_ref[...])


def _layer2_body(feat_ref, p0_ref, p1_ref, w2a_ref, w2b_ref, b2_ref, out_ref):
    ns = p0_ref[...] + p1_ref[...]
    h2 = jnp.dot(feat_ref[...], w2a_ref[...],
                 preferred_element_type=jnp.float32)
    h2 = h2 + jnp.dot(ns, w2b_ref[...], preferred_element_type=jnp.float32)
    h2 = h2 + b2_ref[...]
    nrm = jnp.sqrt(jnp.sum(h2 * h2, axis=1, keepdims=True))
    out_ref[...] = h2 / jnp.maximum(nrm, 1e-12)


_LBLK = 1024


def _layer(body, feat_pad, p0, p1, wa, wb, br):
    grid = (_N_PAD // _LBLK,)
    row_spec = pl.BlockSpec((_LBLK, _D), lambda i: (i, 0))
    full_spec = pl.BlockSpec((_D, _D), lambda i: (0, 0))
    bias_spec = pl.BlockSpec((1, _D), lambda i: (0, 0))
    return pl.pallas_call(
        body,
        grid=grid,
        in_specs=[row_spec, row_spec, row_spec, full_spec, full_spec,
                  bias_spec],
        out_specs=row_spec,
        out_shape=jax.ShapeDtypeStruct((_N_PAD, _D), jnp.float32),
    )(feat_pad, p0, p1, wa, wb, br)


def kernel(feat_data, edge_index, inputs1, inputs2, neg, W1, b1, W2, b2):
    feat_pad = jnp.zeros((_N_PAD, _D), jnp.float32).at[:_N].set(feat_data)
    src = edge_index[0].astype(jnp.int32)
    dst = edge_index[1].astype(jnp.int32)
    pad_e = _E_PAD - _E
    fill = jnp.arange(pad_e, dtype=jnp.int32)
    src_pad = jnp.concatenate(
        [src, fill % _N]).reshape(_NCHUNK, _CHUNK)
    dst_pad = jnp.concatenate(
        [dst, _N + fill % (_N_PAD - _N)]).reshape(_NCHUNK, _CHUNK)

    idx = jnp.concatenate([
        inputs1.astype(jnp.int32), inputs2.astype(jnp.int32),
        neg.astype(jnp.int32),
        jnp.zeros((_G - 2 * _B - _NEG,), jnp.int32)]).reshape(_NS, 2, _GCHUNK)

    p0, p1, gf = _segsum(feat_pad, src_pad, dst_pad, idx)

    w1t = W1.T
    h1 = _layer(_layer1_body, feat_pad, p0, p1, w1t[:_D], w1t[_D:],
                b1.reshape(1, _D))

    gp = _segsum_gather(h1, src_pad, dst_pad, idx)
    g0, g1 = gp[0], gp[1]

    w2t = W2.T
    out = pl.pallas_call(
        _layer2_body,
        out_shape=jax.ShapeDtypeStruct((_G, _D), jnp.float32),
    )(gf, g0, g1, w2t[:_D], w2t[_D:], b2.reshape(1, _D))
    return (out[:_B], out[_B:2 * _B], out[2 * _B:2 * _B + _NEG])
